# baseline (speedup 1.0000x reference)
import functools

import jax
import jax.numpy as jnp
from jax import lax
from jax.experimental import pallas as pl
from jax.experimental.pallas import tpu as pltpu
from jax.experimental.pallas import tpu_sc as plsc

LANES = 16
NBUF = 3


@functools.lru_cache(maxsize=None)
def _make_emb_kernel(batch, seq, vocab, d_model):
    info = plsc.get_sparse_core_info()
    nc, ns = info.num_cores, info.num_subcores
    nw = nc * ns
    assert seq % nw == 0
    s_per_w = seq // nw
    ch = 32
    n_halves = s_per_w // ch
    n_chunks = n_halves * batch
    assert d_model % LANES == 0
    cols = d_model // LANES

    mesh = plsc.VectorSubcoreMesh(core_axis_name="c", subcore_axis_name="s")

    @functools.partial(
        pl.kernel,
        mesh=mesh,
        out_type=jax.ShapeDtypeStruct((batch, seq, d_model), jnp.float32),
        scratch_types=(
            [pltpu.VMEM((ch,), jnp.int32) for _ in range(n_halves * batch)]
            + [pltpu.VMEM((ch, d_model), jnp.float32) for _ in range(NBUF)]
            + [pltpu.VMEM((ch, d_model), jnp.float32) for _ in range(n_halves)]
            + [pltpu.SemaphoreType.DMA for _ in range(2 * NBUF + n_halves + 1)]
        ),
    )
    def emb(idx_hbm, tok_hbm, pos_hbm, out_hbm, *refs):
        nci = n_halves * batch
        idx_v = refs[0:nci]
        tok_v = refs[nci:nci + NBUF]
        pos_v = refs[nci + NBUF:nci + NBUF + n_halves]
        gsem = refs[nci + NBUF + n_halves:nci + 2 * NBUF + n_halves]
        wsem = refs[nci + 2 * NBUF + n_halves:nci + 3 * NBUF + n_halves]
        psem = refs[nci + 3 * NBUF + n_halves:nci + 3 * NBUF + 2 * n_halves]
        isem = refs[nci + 3 * NBUF + 2 * n_halves]

        wid = lax.axis_index("s") * nc + lax.axis_index("c")
        s0 = wid * s_per_w

        def row0_of(g):
            h, k = divmod(g, batch)
            return k * seq + s0 + h * ch

        def issue_gather(g):
            b = g % NBUF
            return pltpu.async_copy(tok_hbm.at[idx_v[g]], tok_v[b], gsem[b])

        def issue_out(g):
            b = g % NBUF
            h, k = divmod(g, batch)
            return pltpu.async_copy(
                tok_v[b], out_hbm.at[k, pl.ds(s0 + h * ch, ch)], wsem[b]
            )

        idx_cp = [
            pltpu.async_copy(
                idx_hbm.at[pl.ds(row0_of(g), ch)], idx_v[g], isem
            )
            for g in range(n_chunks)
        ]
        pos_cp = [
            pltpu.async_copy(
                pos_hbm.at[pl.ds(s0 + h * ch, ch)], pos_v[h], psem[h]
            )
            for h in range(n_halves)
        ]
        for c in idx_cp:
            c.wait()

        def add_chunk(g):
            b = g % NBUF
            h = g // batch

            def row_body(r, carry):
                for c in range(cols):
                    s = c * LANES
                    plsc.addupdate(
                        tok_v[b].at[r, pl.ds(s, LANES)],
                        pos_v[h][r, pl.ds(s, LANES)],
                    )
                return carry

            lax.fori_loop(0, ch, row_body, 0)

        pref = NBUF - 1
        gather_cp = {g: issue_gather(g) for g in range(min(pref, n_chunks))}
        for h in range(n_halves):
            pos_cp[h].wait()
        out_cp = {}
        for g in range(n_chunks):
            gather_cp[g].wait()
            add_chunk(g)
            out_cp[g] = issue_out(g)
            if g + pref < n_chunks:
                if g - 1 >= 0:
                    out_cp[g - 1].wait()
                gather_cp[g + pref] = issue_gather(g + pref)
        for g in range(max(0, n_chunks - NBUF), n_chunks):
            out_cp[g].wait()

    return emb


def kernel(x, token_table, pos_table):
    b, s = x.shape
    v, d = token_table.shape
    idx = x.reshape(b * s).astype(jnp.int32)
    emb = _make_emb_kernel(b, s, v, d)
    return emb(idx, token_table, pos_table)


# speedup vs baseline: 1.0059x; 1.0059x over previous
import functools

import jax
import jax.numpy as jnp
from jax import lax
from jax.experimental import pallas as pl
from jax.experimental.pallas import tpu as pltpu
from jax.experimental.pallas import tpu_sc as plsc

LANES = 16
NBUF = 3


@functools.lru_cache(maxsize=None)
def _make_emb_kernel(batch, seq, vocab, d_model):
    info = plsc.get_sparse_core_info()
    nc, ns = info.num_cores, info.num_subcores
    nw = nc * ns
    assert seq % nw == 0
    s_per_w = seq // nw
    ch = 32
    n_halves = s_per_w // ch
    n_chunks = n_halves * batch
    assert d_model % LANES == 0
    cols = d_model // LANES

    mesh = plsc.VectorSubcoreMesh(core_axis_name="c", subcore_axis_name="s")

    @functools.partial(
        pl.kernel,
        mesh=mesh,
        out_type=jax.ShapeDtypeStruct((batch, seq, d_model), jnp.float32),
        scratch_types=(
            [pltpu.VMEM((ch,), jnp.int32) for _ in range(n_halves * batch)]
            + [pltpu.VMEM((ch, d_model), jnp.float32) for _ in range(NBUF)]
            + [pltpu.VMEM((ch, d_model), jnp.float32) for _ in range(n_halves)]
            + [pltpu.SemaphoreType.DMA for _ in range(2 * NBUF + n_halves + 1)]
        ),
    )
    def emb(x_hbm, tok_hbm, pos_hbm, out_hbm, *refs):
        nci = n_halves * batch
        idx_v = refs[0:nci]
        tok_v = refs[nci:nci + NBUF]
        pos_v = refs[nci + NBUF:nci + NBUF + n_halves]
        gsem = refs[nci + NBUF + n_halves:nci + 2 * NBUF + n_halves]
        wsem = refs[nci + 2 * NBUF + n_halves:nci + 3 * NBUF + n_halves]
        psem = refs[nci + 3 * NBUF + n_halves:nci + 3 * NBUF + 2 * n_halves]
        isem = refs[nci + 3 * NBUF + 2 * n_halves]

        wid = lax.axis_index("s") * nc + lax.axis_index("c")
        s0 = wid * s_per_w

        def row0_of(g):
            h, k = divmod(g, batch)
            return k * seq + s0 + h * ch

        def issue_gather(g):
            b = g % NBUF
            return pltpu.async_copy(tok_hbm.at[idx_v[g]], tok_v[b], gsem[b])

        def issue_out(g):
            b = g % NBUF
            h, k = divmod(g, batch)
            return pltpu.async_copy(
                tok_v[b], out_hbm.at[k, pl.ds(s0 + h * ch, ch)], wsem[b]
            )

        idx_cp = [
            pltpu.async_copy(
                x_hbm.at[g % batch, pl.ds(s0 + (g // batch) * ch, ch)],
                idx_v[g], isem,
            )
            for g in range(n_chunks)
        ]
        pos_cp = [
            pltpu.async_copy(
                pos_hbm.at[pl.ds(s0 + h * ch, ch)], pos_v[h], psem[h]
            )
            for h in range(n_halves)
        ]
        for c in idx_cp:
            c.wait()

        def add_chunk(g):
            b = g % NBUF
            h = g // batch

            def row_body(r, carry):
                for c in range(cols):
                    s = c * LANES
                    plsc.addupdate(
                        tok_v[b].at[r, pl.ds(s, LANES)],
                        pos_v[h][r, pl.ds(s, LANES)],
                    )
                return carry

            lax.fori_loop(0, ch, row_body, 0)

        pref = NBUF - 1
        gather_cp = {g: issue_gather(g) for g in range(min(pref, n_chunks))}
        for h in range(n_halves):
            pos_cp[h].wait()
        out_cp = {}
        for g in range(n_chunks):
            gather_cp[g].wait()
            add_chunk(g)
            out_cp[g] = issue_out(g)
            if g + pref < n_chunks:
                if g - 1 >= 0:
                    out_cp[g - 1].wait()
                gather_cp[g + pref] = issue_gather(g + pref)
        for g in range(max(0, n_chunks - NBUF), n_chunks):
            out_cp[g].wait()

    return emb


def kernel(x, token_table, pos_table):
    b, s = x.shape
    v, d = token_table.shape
    emb = _make_emb_kernel(b, s, v, d)
    return emb(x.astype(jnp.int32), token_table, pos_table)
